# baseline (speedup 1.0000x reference)
import functools

import jax
import jax.numpy as jnp
from jax import lax
from jax.experimental import pallas as pl
from jax.experimental.pallas import tpu as pltpu
from jax.experimental.pallas import tpu_sc as plsc

KNN = 20
EPSILON = 1e-5
ROW_TILE = 512
PT_TILE = 128
NEG_BIG = -1e30
NUM_WORKERS = 32


def _topk_y_and_outputs(n_pts, base, xr, xa, wa_ref, b_ref,
                        idx_ref, y_ref, x2_ref, c_pad):
    xrb = xr.astype(jnp.bfloat16)
    xab = xa.astype(jnp.bfloat16)
    dot = lax.dot_general(xrb, xab, (((1,), (1,)), ((), ())),
                          preferred_element_type=jnp.float32)
    sq_r = jnp.sum(xr * xr, axis=1, keepdims=True)
    sq_a = jnp.sum(xa * xa, axis=1)[None, :]
    d = 2.0 * dot - sq_r - sq_a
    n_iota = lax.broadcasted_iota(jnp.int32, d.shape, 1)
    cols = []
    for _ in range(KNN):
        m = jnp.max(d, axis=1, keepdims=True)
        cand = jnp.where(d == m, n_iota, n_pts)
        jmin = jnp.min(cand, axis=1)
        cols.append(jmin[None, :] + base)
        d = jnp.where(n_iota == jmin[:, None], NEG_BIG, d)
    idx_ref[0] = jnp.concatenate(cols, axis=0)
    y_ref[0] = lax.dot_general(
        xrb, wa_ref[...].astype(jnp.bfloat16), (((1,), (0,)), ((), ())),
        preferred_element_type=jnp.float32) + b_ref[...]
    pad = c_pad - xr.shape[1]
    if pad:
        x2_ref[0] = jnp.concatenate(
            [xr, jnp.zeros((xr.shape[0], pad), jnp.float32)], axis=1)
    else:
        x2_ref[0] = xr


def _stage_a1_body(n_pts, c_pad, xr_ref, xa_ref, wa_ref, b_ref,
                   idx_ref, y_ref, x2_ref):
    base = pl.program_id(0) * n_pts
    _topk_y_and_outputs(n_pts, base, xr_ref[0], xa_ref[0], wa_ref, b_ref,
                        idx_ref, y_ref, x2_ref, c_pad)


def _stage_a1(xt, wa, bias, c_pad):
    bsz, n_pts, c_in = xt.shape
    g_out = wa.shape[1]
    r = ROW_TILE
    return pl.pallas_call(
        functools.partial(_stage_a1_body, n_pts, c_pad),
        grid=(bsz, n_pts // r),
        in_specs=[
            pl.BlockSpec((1, r, c_in), lambda b, t: (b, t, 0)),
            pl.BlockSpec((1, n_pts, c_in), lambda b, t: (b, 0, 0)),
            pl.BlockSpec((c_in, g_out), lambda b, t: (0, 0)),
            pl.BlockSpec((1, g_out), lambda b, t: (0, 0)),
        ],
        out_specs=[
            pl.BlockSpec((1, KNN, r), lambda b, t: (b, 0, t)),
            pl.BlockSpec((1, r, g_out), lambda b, t: (b, t, 0)),
            pl.BlockSpec((1, r, c_pad), lambda b, t: (b, t, 0)),
        ],
        out_shape=[
            jax.ShapeDtypeStruct((bsz, KNN, n_pts), jnp.int32),
            jax.ShapeDtypeStruct((bsz, n_pts, g_out), jnp.float32),
            jax.ShapeDtypeStruct((bsz, n_pts, c_pad), jnp.float32),
        ],
    )(xt, xt, wa, bias)


def _bn_finish(q, p1_ref, p2_ref, g_ref, be_ref, n_total):
    cnt = float(KNN) * n_total
    mean = jnp.sum(p1_ref[...], axis=0) / cnt
    e2 = jnp.sum(p2_ref[...], axis=0) / cnt
    var = e2 - mean * mean
    scale = lax.rsqrt(var + EPSILON) * g_ref[...]
    return jnp.maximum((q - mean) * scale + be_ref[...], 0.0)


def _stage_a23_body(n_pts, n_total, c_prev, c_pad, xp_r_ref, xp_a_ref,
                    q_r_ref, q_a_ref, p1_ref, p2_ref, g_ref, be_ref,
                    wa_ref, b_ref, idx_ref, y_ref, x2_ref):
    base = pl.program_id(0) * n_pts
    f_a = _bn_finish(q_a_ref[0], p1_ref, p2_ref, g_ref, be_ref, n_total)
    f_r = _bn_finish(q_r_ref[0], p1_ref, p2_ref, g_ref, be_ref, n_total)
    xa = jnp.concatenate([f_a, xp_a_ref[0][:, :c_prev]], axis=1)
    xr = jnp.concatenate([f_r, xp_r_ref[0][:, :c_prev]], axis=1)
    _topk_y_and_outputs(n_pts, base, xr, xa, wa_ref, b_ref,
                        idx_ref, y_ref, x2_ref, c_pad)


def _stage_a23(x2_prev, q, p1, p2, gamma, beta, wa, bias, c_prev, c_pad):
    bsz, n_pts, cp_prev = x2_prev.shape
    g_out = q.shape[-1]
    n_total = bsz * n_pts
    r = ROW_TILE
    n_tiles = p1.shape[0]
    return pl.pallas_call(
        functools.partial(_stage_a23_body, n_pts, n_total, c_prev, c_pad),
        grid=(bsz, n_pts // r),
        in_specs=[
            pl.BlockSpec((1, r, cp_prev), lambda b, t: (b, t, 0)),
            pl.BlockSpec((1, n_pts, cp_prev), lambda b, t: (b, 0, 0)),
            pl.BlockSpec((1, r, g_out), lambda b, t: (b, t, 0)),
            pl.BlockSpec((1, n_pts, g_out), lambda b, t: (b, 0, 0)),
            pl.BlockSpec((n_tiles, 1, g_out), lambda b, t: (0, 0, 0)),
            pl.BlockSpec((n_tiles, 1, g_out), lambda b, t: (0, 0, 0)),
            pl.BlockSpec((1, g_out), lambda b, t: (0, 0)),
            pl.BlockSpec((1, g_out), lambda b, t: (0, 0)),
            pl.BlockSpec((c_prev + g_out, g_out), lambda b, t: (0, 0)),
            pl.BlockSpec((1, g_out), lambda b, t: (0, 0)),
        ],
        out_specs=[
            pl.BlockSpec((1, KNN, r), lambda b, t: (b, 0, t)),
            pl.BlockSpec((1, r, g_out), lambda b, t: (b, t, 0)),
            pl.BlockSpec((1, r, c_pad), lambda b, t: (b, t, 0)),
        ],
        out_shape=[
            jax.ShapeDtypeStruct((bsz, KNN, n_pts), jnp.int32),
            jax.ShapeDtypeStruct((bsz, n_pts, g_out), jnp.float32),
            jax.ShapeDtypeStruct((bsz, n_pts, c_pad), jnp.float32),
        ],
    )(x2_prev, x2_prev, q, q, p1, p2, gamma, beta, wa, bias)


def _make_stage_b(n_total, c_pad):
    rows = (n_total * KNN) // NUM_WORKERS
    nch = rows // 128
    mesh = plsc.VectorSubcoreMesh(core_axis_name="c", subcore_axis_name="s")

    @functools.partial(
        pl.kernel,
        out_type=jax.ShapeDtypeStruct((n_total * KNN, c_pad), jnp.float32),
        mesh=mesh,
        compiler_params=pltpu.CompilerParams(needs_layout_passes=False),
        scratch_types=[
            pltpu.VMEM((rows,), jnp.int32),
            pltpu.VMEM((128, c_pad), jnp.float32),
            pltpu.VMEM((128, c_pad), jnp.float32),
            pltpu.VMEM((128, c_pad), jnp.float32),
            pltpu.SemaphoreType.DMA,
            pltpu.SemaphoreType.DMA,
            pltpu.SemaphoreType.DMA,
            pltpu.SemaphoreType.DMA,
            pltpu.SemaphoreType.DMA,
            pltpu.SemaphoreType.DMA,
        ],
    )
    def stage_b(x_hbm, idx_hbm, xj_hbm, idx_v, buf0, buf1, buf2,
                g0, g1, g2, o0, o1, o2):
        wid = lax.axis_index("s") * 2 + lax.axis_index("c")
        base = wid * rows
        pltpu.sync_copy(idx_hbm.at[wid], idx_v)
        bufs = (buf0, buf1, buf2)
        gsems = (g0, g1, g2)
        osems = (o0, o1, o2)
        pend_out = [None, None, None]
        prev = None
        for ch in range(nch):
            s = ch % 3
            if pend_out[s] is not None:
                pend_out[s].wait()
                pend_out[s] = None
            g = pltpu.async_copy(
                x_hbm.at[idx_v.at[pl.ds(ch * 128, 128)]], bufs[s], gsems[s])
            if prev is not None:
                ps, pg, pch = prev
                pg.wait()
                pend_out[ps] = pltpu.async_copy(
                    bufs[ps], xj_hbm.at[pl.ds(base + pch * 128, 128)],
                    osems[ps])
            prev = (s, g, ch)
        ps, pg, pch = prev
        pg.wait()
        pend_out[ps] = pltpu.async_copy(
            bufs[ps], xj_hbm.at[pl.ds(base + pch * 128, 128)], osems[ps])
        for s in range(3):
            if pend_out[s] is not None:
                pend_out[s].wait()

    return stage_b


def _stage_c_body(xj_ref, xi_ref, y_ref, wb_ref, q_ref, p1_ref, p2_ref):
    pt = xi_ref.shape[0]
    c_pad = xi_ref.shape[1]
    xj = xj_ref[...].reshape(pt, KNN, c_pad)
    diff = (xj - xi_ref[...][:, None, :]).astype(jnp.bfloat16)
    wb = wb_ref[...].astype(jnp.bfloat16)
    gm = lax.dot_general(diff.reshape(pt * KNN, c_pad), wb,
                         (((1,), (0,)), ((), ())),
                         preferred_element_type=jnp.float32)
    gm = gm.reshape(pt, KNN, wb.shape[1])
    y = y_ref[...]
    s1 = jnp.sum(gm, axis=1)
    s2 = jnp.sum(gm * gm, axis=1)
    k = float(KNN)
    q_ref[...] = y + jnp.max(gm, axis=1)
    p1_ref[0] = jnp.sum(k * y + s1, axis=0, keepdims=True)
    p2_ref[0] = jnp.sum(k * y * y + 2.0 * y * s1 + s2, axis=0,
                        keepdims=True)


def _stage_c(xj, x2f, yf, wb_pad):
    n_total, c_pad = x2f.shape
    g_out = yf.shape[1]
    pt = PT_TILE
    n_tiles = n_total // pt
    return pl.pallas_call(
        _stage_c_body,
        grid=(n_tiles,),
        in_specs=[
            pl.BlockSpec((pt * KNN, c_pad), lambda t: (t, 0)),
            pl.BlockSpec((pt, c_pad), lambda t: (t, 0)),
            pl.BlockSpec((pt, g_out), lambda t: (t, 0)),
            pl.BlockSpec((c_pad, g_out), lambda t: (0, 0)),
        ],
        out_specs=[
            pl.BlockSpec((pt, g_out), lambda t: (t, 0)),
            pl.BlockSpec((1, 1, g_out), lambda t: (t, 0, 0)),
            pl.BlockSpec((1, 1, g_out), lambda t: (t, 0, 0)),
        ],
        out_shape=[
            jax.ShapeDtypeStruct((n_total, g_out), jnp.float32),
            jax.ShapeDtypeStruct((n_tiles, 1, g_out), jnp.float32),
            jax.ShapeDtypeStruct((n_tiles, 1, g_out), jnp.float32),
        ],
    )(xj, x2f, yf, wb_pad)


def _stage_f_body(n_total, q_ref, p1_ref, p2_ref, g_ref, be_ref, f_ref):
    f_ref[...] = _bn_finish(q_ref[...], p1_ref, p2_ref, g_ref, be_ref,
                            n_total)


def _stage_f(q, p1, p2, gamma, beta):
    n_total, g_out = q.shape
    return pl.pallas_call(
        functools.partial(_stage_f_body, n_total),
        out_shape=jax.ShapeDtypeStruct((n_total, g_out), jnp.float32),
    )(q, p1, p2, gamma, beta)


def _wab(w, c_in):
    wa = jnp.transpose(w[:, :c_in], (1, 0))
    wb = jnp.transpose(w[:, c_in:], (1, 0))
    return wa, wb


def _gather_and_edges(idx, x2, wb, y):
    bsz, n_pts, c_pad = x2.shape
    g_out = wb.shape[1]
    c_in = wb.shape[0]
    n_total = bsz * n_pts
    wb_pad = jnp.concatenate(
        [wb, jnp.zeros((c_pad - c_in, g_out), jnp.float32)], axis=0)
    idx_w = jnp.transpose(idx, (0, 2, 1)).reshape(
        NUM_WORKERS, (n_total * KNN) // NUM_WORKERS)
    x2f = x2.reshape(n_total, c_pad)
    xj = _make_stage_b(n_total, c_pad)(x2f, idx_w)
    return _stage_c(xj, x2f, y.reshape(n_total, g_out), wb_pad)


def kernel(inputs, W0, b0, g0, be0, W1, b1, g1, be1, W2, b2, g2, be2):
    xt0 = jnp.transpose(inputs[..., 0], (0, 2, 1))
    bsz, n_pts, c0 = xt0.shape
    n_total = bsz * n_pts

    wa0, wb0 = _wab(W0, c0)
    cp1 = 128
    idx0, y0, x2_0 = _stage_a1(xt0, wa0, b0[None, :], cp1)
    q0, p10, p20 = _gather_and_edges(idx0, x2_0, wb0, y0)

    c1 = c0 + 64
    wa1, wb1 = _wab(W1, c1)
    idx1, y1, x2_1 = _stage_a23(x2_0, q0.reshape(bsz, n_pts, -1), p10, p20,
                                g0[None, :], be0[None, :], wa1, b1[None, :],
                                c0, 128)
    q1, p11, p21 = _gather_and_edges(idx1, x2_1, wb1, y1)

    c2 = c1 + 64
    wa2, wb2 = _wab(W2, c2)
    idx2, y2, x2_2 = _stage_a23(x2_1, q1.reshape(bsz, n_pts, -1), p11, p21,
                                g1[None, :], be1[None, :], wa2, b2[None, :],
                                c1, 256)
    q2, p12, p22 = _gather_and_edges(idx2, x2_2, wb2, y2)

    f2 = _stage_f(q2, p12, p22, g2[None, :], be2[None, :])
    out = jnp.concatenate(
        [f2.reshape(bsz, n_pts, 64), x2_2[:, :, :c2]], axis=-1)
    return jnp.transpose(out, (0, 2, 1))[..., None]


# speedup vs baseline: 1.0080x; 1.0080x over previous
import functools

import jax
import jax.numpy as jnp
from jax import lax
from jax.experimental import pallas as pl
from jax.experimental.pallas import tpu as pltpu
from jax.experimental.pallas import tpu_sc as plsc

KNN = 20
EPSILON = 1e-5
ROW_TILE = 1024
PT_TILE = 128
NEG_BIG = -1e30
NUM_WORKERS = 32


def _topk_y_and_outputs(n_pts, base, xr, xa, wa_ref, b_ref,
                        idx_ref, y_ref, x2_ref, c_pad):
    xrb = xr.astype(jnp.bfloat16)
    xab = xa.astype(jnp.bfloat16)
    dot = lax.dot_general(xrb, xab, (((1,), (1,)), ((), ())),
                          preferred_element_type=jnp.float32)
    sq_r = jnp.sum(xr * xr, axis=1, keepdims=True)
    sq_a = jnp.sum(xa * xa, axis=1)[None, :]
    d = 2.0 * dot - sq_r - sq_a
    n_iota = lax.broadcasted_iota(jnp.int32, d.shape, 1)
    cols = []
    for _ in range(KNN):
        m = jnp.max(d, axis=1, keepdims=True)
        cand = jnp.where(d == m, n_iota, n_pts)
        jmin = jnp.min(cand, axis=1)
        cols.append(jmin[None, :] + base)
        d = jnp.where(n_iota == jmin[:, None], NEG_BIG, d)
    idx_ref[0] = jnp.concatenate(cols, axis=0)
    y_ref[0] = lax.dot_general(
        xrb, wa_ref[...].astype(jnp.bfloat16), (((1,), (0,)), ((), ())),
        preferred_element_type=jnp.float32) + b_ref[...]
    pad = c_pad - xr.shape[1]
    if pad:
        x2_ref[0] = jnp.concatenate(
            [xr, jnp.zeros((xr.shape[0], pad), jnp.float32)], axis=1)
    else:
        x2_ref[0] = xr


def _stage_a1_body(n_pts, c_pad, xr_ref, xa_ref, wa_ref, b_ref,
                   idx_ref, y_ref, x2_ref):
    base = pl.program_id(0) * n_pts
    _topk_y_and_outputs(n_pts, base, xr_ref[0], xa_ref[0], wa_ref, b_ref,
                        idx_ref, y_ref, x2_ref, c_pad)


def _stage_a1(xt, wa, bias, c_pad):
    bsz, n_pts, c_in = xt.shape
    g_out = wa.shape[1]
    r = ROW_TILE
    return pl.pallas_call(
        functools.partial(_stage_a1_body, n_pts, c_pad),
        grid=(bsz, n_pts // r),
        in_specs=[
            pl.BlockSpec((1, r, c_in), lambda b, t: (b, t, 0)),
            pl.BlockSpec((1, n_pts, c_in), lambda b, t: (b, 0, 0)),
            pl.BlockSpec((c_in, g_out), lambda b, t: (0, 0)),
            pl.BlockSpec((1, g_out), lambda b, t: (0, 0)),
        ],
        out_specs=[
            pl.BlockSpec((1, KNN, r), lambda b, t: (b, 0, t)),
            pl.BlockSpec((1, r, g_out), lambda b, t: (b, t, 0)),
            pl.BlockSpec((1, r, c_pad), lambda b, t: (b, t, 0)),
        ],
        out_shape=[
            jax.ShapeDtypeStruct((bsz, KNN, n_pts), jnp.int32),
            jax.ShapeDtypeStruct((bsz, n_pts, g_out), jnp.float32),
            jax.ShapeDtypeStruct((bsz, n_pts, c_pad), jnp.float32),
        ],
    )(xt, xt, wa, bias)


def _bn_finish(q, p1_ref, p2_ref, g_ref, be_ref, n_total):
    cnt = float(KNN) * n_total
    mean = jnp.sum(p1_ref[...], axis=0) / cnt
    e2 = jnp.sum(p2_ref[...], axis=0) / cnt
    var = e2 - mean * mean
    scale = lax.rsqrt(var + EPSILON) * g_ref[...]
    return jnp.maximum((q - mean) * scale + be_ref[...], 0.0)


def _stage_a23_body(n_pts, n_total, c_prev, c_pad, xp_r_ref, xp_a_ref,
                    q_r_ref, q_a_ref, p1_ref, p2_ref, g_ref, be_ref,
                    wa_ref, b_ref, idx_ref, y_ref, x2_ref):
    base = pl.program_id(0) * n_pts
    f_a = _bn_finish(q_a_ref[0], p1_ref, p2_ref, g_ref, be_ref, n_total)
    f_r = _bn_finish(q_r_ref[0], p1_ref, p2_ref, g_ref, be_ref, n_total)
    xa = jnp.concatenate([f_a, xp_a_ref[0][:, :c_prev]], axis=1)
    xr = jnp.concatenate([f_r, xp_r_ref[0][:, :c_prev]], axis=1)
    _topk_y_and_outputs(n_pts, base, xr, xa, wa_ref, b_ref,
                        idx_ref, y_ref, x2_ref, c_pad)


def _stage_a23(x2_prev, q, p1, p2, gamma, beta, wa, bias, c_prev, c_pad):
    bsz, n_pts, cp_prev = x2_prev.shape
    g_out = q.shape[-1]
    n_total = bsz * n_pts
    r = ROW_TILE
    n_tiles = p1.shape[0]
    return pl.pallas_call(
        functools.partial(_stage_a23_body, n_pts, n_total, c_prev, c_pad),
        grid=(bsz, n_pts // r),
        in_specs=[
            pl.BlockSpec((1, r, cp_prev), lambda b, t: (b, t, 0)),
            pl.BlockSpec((1, n_pts, cp_prev), lambda b, t: (b, 0, 0)),
            pl.BlockSpec((1, r, g_out), lambda b, t: (b, t, 0)),
            pl.BlockSpec((1, n_pts, g_out), lambda b, t: (b, 0, 0)),
            pl.BlockSpec((n_tiles, 1, g_out), lambda b, t: (0, 0, 0)),
            pl.BlockSpec((n_tiles, 1, g_out), lambda b, t: (0, 0, 0)),
            pl.BlockSpec((1, g_out), lambda b, t: (0, 0)),
            pl.BlockSpec((1, g_out), lambda b, t: (0, 0)),
            pl.BlockSpec((c_prev + g_out, g_out), lambda b, t: (0, 0)),
            pl.BlockSpec((1, g_out), lambda b, t: (0, 0)),
        ],
        out_specs=[
            pl.BlockSpec((1, KNN, r), lambda b, t: (b, 0, t)),
            pl.BlockSpec((1, r, g_out), lambda b, t: (b, t, 0)),
            pl.BlockSpec((1, r, c_pad), lambda b, t: (b, t, 0)),
        ],
        out_shape=[
            jax.ShapeDtypeStruct((bsz, KNN, n_pts), jnp.int32),
            jax.ShapeDtypeStruct((bsz, n_pts, g_out), jnp.float32),
            jax.ShapeDtypeStruct((bsz, n_pts, c_pad), jnp.float32),
        ],
    )(x2_prev, x2_prev, q, q, p1, p2, gamma, beta, wa, bias)


def _make_stage_b(n_total, c_pad):
    rows = (n_total * KNN) // NUM_WORKERS
    nch = rows // 128
    mesh = plsc.VectorSubcoreMesh(core_axis_name="c", subcore_axis_name="s")

    @functools.partial(
        pl.kernel,
        out_type=jax.ShapeDtypeStruct((n_total * KNN, c_pad), jnp.float32),
        mesh=mesh,
        compiler_params=pltpu.CompilerParams(needs_layout_passes=False),
        scratch_types=[
            pltpu.VMEM((rows,), jnp.int32),
            pltpu.VMEM((128, c_pad), jnp.float32),
            pltpu.VMEM((128, c_pad), jnp.float32),
            pltpu.VMEM((128, c_pad), jnp.float32),
            pltpu.SemaphoreType.DMA,
            pltpu.SemaphoreType.DMA,
            pltpu.SemaphoreType.DMA,
            pltpu.SemaphoreType.DMA,
            pltpu.SemaphoreType.DMA,
            pltpu.SemaphoreType.DMA,
        ],
    )
    def stage_b(x_hbm, idx_hbm, xj_hbm, idx_v, buf0, buf1, buf2,
                g0, g1, g2, o0, o1, o2):
        wid = lax.axis_index("s") * 2 + lax.axis_index("c")
        base = wid * rows
        pltpu.sync_copy(idx_hbm.at[wid], idx_v)
        bufs = (buf0, buf1, buf2)
        gsems = (g0, g1, g2)
        osems = (o0, o1, o2)
        pend_out = [None, None, None]
        prev = None
        for ch in range(nch):
            s = ch % 3
            if pend_out[s] is not None:
                pend_out[s].wait()
                pend_out[s] = None
            g = pltpu.async_copy(
                x_hbm.at[idx_v.at[pl.ds(ch * 128, 128)]], bufs[s], gsems[s])
            if prev is not None:
                ps, pg, pch = prev
                pg.wait()
                pend_out[ps] = pltpu.async_copy(
                    bufs[ps], xj_hbm.at[pl.ds(base + pch * 128, 128)],
                    osems[ps])
            prev = (s, g, ch)
        ps, pg, pch = prev
        pg.wait()
        pend_out[ps] = pltpu.async_copy(
            bufs[ps], xj_hbm.at[pl.ds(base + pch * 128, 128)], osems[ps])
        for s in range(3):
            if pend_out[s] is not None:
                pend_out[s].wait()

    return stage_b


def _stage_c_body(xj_ref, xi_ref, y_ref, wb_ref, q_ref, p1_ref, p2_ref):
    pt = xi_ref.shape[0]
    c_pad = xi_ref.shape[1]
    xj = xj_ref[...].reshape(pt, KNN, c_pad)
    diff = (xj - xi_ref[...][:, None, :]).astype(jnp.bfloat16)
    wb = wb_ref[...].astype(jnp.bfloat16)
    gm = lax.dot_general(diff.reshape(pt * KNN, c_pad), wb,
                         (((1,), (0,)), ((), ())),
                         preferred_element_type=jnp.float32)
    gm = gm.reshape(pt, KNN, wb.shape[1])
    y = y_ref[...]
    s1 = jnp.sum(gm, axis=1)
    s2 = jnp.sum(gm * gm, axis=1)
    k = float(KNN)
    q_ref[...] = y + jnp.max(gm, axis=1)
    p1_ref[0] = jnp.sum(k * y + s1, axis=0, keepdims=True)
    p2_ref[0] = jnp.sum(k * y * y + 2.0 * y * s1 + s2, axis=0,
                        keepdims=True)


def _stage_c(xj, x2f, yf, wb_pad):
    n_total, c_pad = x2f.shape
    g_out = yf.shape[1]
    pt = PT_TILE
    n_tiles = n_total // pt
    return pl.pallas_call(
        _stage_c_body,
        grid=(n_tiles,),
        in_specs=[
            pl.BlockSpec((pt * KNN, c_pad), lambda t: (t, 0)),
            pl.BlockSpec((pt, c_pad), lambda t: (t, 0)),
            pl.BlockSpec((pt, g_out), lambda t: (t, 0)),
            pl.BlockSpec((c_pad, g_out), lambda t: (0, 0)),
        ],
        out_specs=[
            pl.BlockSpec((pt, g_out), lambda t: (t, 0)),
            pl.BlockSpec((1, 1, g_out), lambda t: (t, 0, 0)),
            pl.BlockSpec((1, 1, g_out), lambda t: (t, 0, 0)),
        ],
        out_shape=[
            jax.ShapeDtypeStruct((n_total, g_out), jnp.float32),
            jax.ShapeDtypeStruct((n_tiles, 1, g_out), jnp.float32),
            jax.ShapeDtypeStruct((n_tiles, 1, g_out), jnp.float32),
        ],
    )(xj, x2f, yf, wb_pad)


def _stage_f_body(n_total, q_ref, p1_ref, p2_ref, g_ref, be_ref, f_ref):
    f_ref[...] = _bn_finish(q_ref[...], p1_ref, p2_ref, g_ref, be_ref,
                            n_total)


def _stage_f(q, p1, p2, gamma, beta):
    n_total, g_out = q.shape
    return pl.pallas_call(
        functools.partial(_stage_f_body, n_total),
        out_shape=jax.ShapeDtypeStruct((n_total, g_out), jnp.float32),
    )(q, p1, p2, gamma, beta)


def _wab(w, c_in):
    wa = jnp.transpose(w[:, :c_in], (1, 0))
    wb = jnp.transpose(w[:, c_in:], (1, 0))
    return wa, wb


def _gather_and_edges(idx, x2, wb, y):
    bsz, n_pts, c_pad = x2.shape
    g_out = wb.shape[1]
    c_in = wb.shape[0]
    n_total = bsz * n_pts
    wb_pad = jnp.concatenate(
        [wb, jnp.zeros((c_pad - c_in, g_out), jnp.float32)], axis=0)
    idx_w = jnp.transpose(idx, (0, 2, 1)).reshape(
        NUM_WORKERS, (n_total * KNN) // NUM_WORKERS)
    x2f = x2.reshape(n_total, c_pad)
    xj = _make_stage_b(n_total, c_pad)(x2f, idx_w)
    return _stage_c(xj, x2f, y.reshape(n_total, g_out), wb_pad)


def kernel(inputs, W0, b0, g0, be0, W1, b1, g1, be1, W2, b2, g2, be2):
    xt0 = jnp.transpose(inputs[..., 0], (0, 2, 1))
    bsz, n_pts, c0 = xt0.shape
    n_total = bsz * n_pts

    wa0, wb0 = _wab(W0, c0)
    cp1 = 128
    idx0, y0, x2_0 = _stage_a1(xt0, wa0, b0[None, :], cp1)
    q0, p10, p20 = _gather_and_edges(idx0, x2_0, wb0, y0)

    c1 = c0 + 64
    wa1, wb1 = _wab(W1, c1)
    idx1, y1, x2_1 = _stage_a23(x2_0, q0.reshape(bsz, n_pts, -1), p10, p20,
                                g0[None, :], be0[None, :], wa1, b1[None, :],
                                c0, 128)
    q1, p11, p21 = _gather_and_edges(idx1, x2_1, wb1, y1)

    c2 = c1 + 64
    wa2, wb2 = _wab(W2, c2)
    idx2, y2, x2_2 = _stage_a23(x2_1, q1.reshape(bsz, n_pts, -1), p11, p21,
                                g1[None, :], be1[None, :], wa2, b2[None, :],
                                c1, 256)
    q2, p12, p22 = _gather_and_edges(idx2, x2_2, wb2, y2)

    f2 = _stage_f(q2, p12, p22, g2[None, :], be2[None, :])
    out = jnp.concatenate(
        [f2.reshape(bsz, n_pts, 64), x2_2[:, :, :c2]], axis=-1)
    return jnp.transpose(out, (0, 2, 1))[..., None]


# speedup vs baseline: 1.0205x; 1.0124x over previous
import functools

import jax
import jax.numpy as jnp
from jax import lax
from jax.experimental import pallas as pl
from jax.experimental.pallas import tpu as pltpu
from jax.experimental.pallas import tpu_sc as plsc

KNN = 20
EPSILON = 1e-5
ROW_TILE = 1024
PT_TILE = 256
NEG_BIG = -1e30
NUM_WORKERS = 32


def _topk_y_and_outputs(n_pts, base, xr, xa, wa_ref, b_ref,
                        idx_ref, y_ref, x2_ref, c_pad):
    xrb = xr.astype(jnp.bfloat16)
    xab = xa.astype(jnp.bfloat16)
    dot = lax.dot_general(xrb, xab, (((1,), (1,)), ((), ())),
                          preferred_element_type=jnp.float32)
    sq_r = jnp.sum(xr * xr, axis=1, keepdims=True)
    sq_a = jnp.sum(xa * xa, axis=1)[None, :]
    d = 2.0 * dot - sq_r - sq_a
    n_iota = lax.broadcasted_iota(jnp.int32, d.shape, 1)
    cols = []
    for _ in range(KNN):
        m = jnp.max(d, axis=1, keepdims=True)
        cand = jnp.where(d == m, n_iota, n_pts)
        jmin = jnp.min(cand, axis=1)
        cols.append(jmin[None, :] + base)
        d = jnp.where(n_iota == jmin[:, None], NEG_BIG, d)
    idx_ref[0] = jnp.concatenate(cols, axis=0)
    y_ref[0] = lax.dot_general(
        xrb, wa_ref[...].astype(jnp.bfloat16), (((1,), (0,)), ((), ())),
        preferred_element_type=jnp.float32) + b_ref[...]
    pad = c_pad - xr.shape[1]
    if pad:
        x2_ref[0] = jnp.concatenate(
            [xr, jnp.zeros((xr.shape[0], pad), jnp.float32)], axis=1)
    else:
        x2_ref[0] = xr


def _stage_a1_body(n_pts, c_pad, xr_ref, xa_ref, wa_ref, b_ref,
                   idx_ref, y_ref, x2_ref):
    base = pl.program_id(0) * n_pts
    _topk_y_and_outputs(n_pts, base, xr_ref[0], xa_ref[0], wa_ref, b_ref,
                        idx_ref, y_ref, x2_ref, c_pad)


def _stage_a1(xt, wa, bias, c_pad):
    bsz, n_pts, c_in = xt.shape
    g_out = wa.shape[1]
    r = ROW_TILE
    return pl.pallas_call(
        functools.partial(_stage_a1_body, n_pts, c_pad),
        grid=(bsz, n_pts // r),
        in_specs=[
            pl.BlockSpec((1, r, c_in), lambda b, t: (b, t, 0)),
            pl.BlockSpec((1, n_pts, c_in), lambda b, t: (b, 0, 0)),
            pl.BlockSpec((c_in, g_out), lambda b, t: (0, 0)),
            pl.BlockSpec((1, g_out), lambda b, t: (0, 0)),
        ],
        out_specs=[
            pl.BlockSpec((1, KNN, r), lambda b, t: (b, 0, t)),
            pl.BlockSpec((1, r, g_out), lambda b, t: (b, t, 0)),
            pl.BlockSpec((1, r, c_pad), lambda b, t: (b, t, 0)),
        ],
        out_shape=[
            jax.ShapeDtypeStruct((bsz, KNN, n_pts), jnp.int32),
            jax.ShapeDtypeStruct((bsz, n_pts, g_out), jnp.float32),
            jax.ShapeDtypeStruct((bsz, n_pts, c_pad), jnp.float32),
        ],
    )(xt, xt, wa, bias)


def _bn_finish(q, p1_ref, p2_ref, g_ref, be_ref, n_total):
    cnt = float(KNN) * n_total
    mean = jnp.sum(p1_ref[...], axis=0) / cnt
    e2 = jnp.sum(p2_ref[...], axis=0) / cnt
    var = e2 - mean * mean
    scale = lax.rsqrt(var + EPSILON) * g_ref[...]
    return jnp.maximum((q - mean) * scale + be_ref[...], 0.0)


def _stage_a23_body(n_pts, n_total, c_prev, c_pad, xp_r_ref, xp_a_ref,
                    q_r_ref, q_a_ref, p1_ref, p2_ref, g_ref, be_ref,
                    wa_ref, b_ref, idx_ref, y_ref, x2_ref):
    base = pl.program_id(0) * n_pts
    f_a = _bn_finish(q_a_ref[0], p1_ref, p2_ref, g_ref, be_ref, n_total)
    f_r = _bn_finish(q_r_ref[0], p1_ref, p2_ref, g_ref, be_ref, n_total)
    xa = jnp.concatenate([f_a, xp_a_ref[0][:, :c_prev]], axis=1)
    xr = jnp.concatenate([f_r, xp_r_ref[0][:, :c_prev]], axis=1)
    _topk_y_and_outputs(n_pts, base, xr, xa, wa_ref, b_ref,
                        idx_ref, y_ref, x2_ref, c_pad)


def _stage_a23(x2_prev, q, p1, p2, gamma, beta, wa, bias, c_prev, c_pad):
    bsz, n_pts, cp_prev = x2_prev.shape
    g_out = q.shape[-1]
    n_total = bsz * n_pts
    r = ROW_TILE
    n_tiles = p1.shape[0]
    return pl.pallas_call(
        functools.partial(_stage_a23_body, n_pts, n_total, c_prev, c_pad),
        grid=(bsz, n_pts // r),
        in_specs=[
            pl.BlockSpec((1, r, cp_prev), lambda b, t: (b, t, 0)),
            pl.BlockSpec((1, n_pts, cp_prev), lambda b, t: (b, 0, 0)),
            pl.BlockSpec((1, r, g_out), lambda b, t: (b, t, 0)),
            pl.BlockSpec((1, n_pts, g_out), lambda b, t: (b, 0, 0)),
            pl.BlockSpec((n_tiles, 1, g_out), lambda b, t: (0, 0, 0)),
            pl.BlockSpec((n_tiles, 1, g_out), lambda b, t: (0, 0, 0)),
            pl.BlockSpec((1, g_out), lambda b, t: (0, 0)),
            pl.BlockSpec((1, g_out), lambda b, t: (0, 0)),
            pl.BlockSpec((c_prev + g_out, g_out), lambda b, t: (0, 0)),
            pl.BlockSpec((1, g_out), lambda b, t: (0, 0)),
        ],
        out_specs=[
            pl.BlockSpec((1, KNN, r), lambda b, t: (b, 0, t)),
            pl.BlockSpec((1, r, g_out), lambda b, t: (b, t, 0)),
            pl.BlockSpec((1, r, c_pad), lambda b, t: (b, t, 0)),
        ],
        out_shape=[
            jax.ShapeDtypeStruct((bsz, KNN, n_pts), jnp.int32),
            jax.ShapeDtypeStruct((bsz, n_pts, g_out), jnp.float32),
            jax.ShapeDtypeStruct((bsz, n_pts, c_pad), jnp.float32),
        ],
    )(x2_prev, x2_prev, q, q, p1, p2, gamma, beta, wa, bias)


def _make_stage_b(n_total, c_pad):
    rows = (n_total * KNN) // NUM_WORKERS
    nch = rows // 128
    mesh = plsc.VectorSubcoreMesh(core_axis_name="c", subcore_axis_name="s")

    @functools.partial(
        pl.kernel,
        out_type=jax.ShapeDtypeStruct((n_total * KNN, c_pad), jnp.float32),
        mesh=mesh,
        compiler_params=pltpu.CompilerParams(needs_layout_passes=False),
        scratch_types=[
            pltpu.VMEM((rows,), jnp.int32),
            pltpu.VMEM((128, c_pad), jnp.float32),
            pltpu.VMEM((128, c_pad), jnp.float32),
            pltpu.VMEM((128, c_pad), jnp.float32),
            pltpu.SemaphoreType.DMA,
            pltpu.SemaphoreType.DMA,
            pltpu.SemaphoreType.DMA,
            pltpu.SemaphoreType.DMA,
            pltpu.SemaphoreType.DMA,
            pltpu.SemaphoreType.DMA,
        ],
    )
    def stage_b(x_hbm, idx_hbm, xj_hbm, idx_v, buf0, buf1, buf2,
                g0, g1, g2, o0, o1, o2):
        wid = lax.axis_index("s") * 2 + lax.axis_index("c")
        base = wid * rows
        pltpu.sync_copy(idx_hbm.at[wid], idx_v)
        bufs = (buf0, buf1, buf2)
        gsems = (g0, g1, g2)
        osems = (o0, o1, o2)
        pend_out = [None, None, None]
        prev = None
        for ch in range(nch):
            s = ch % 3
            if pend_out[s] is not None:
                pend_out[s].wait()
                pend_out[s] = None
            g = pltpu.async_copy(
                x_hbm.at[idx_v.at[pl.ds(ch * 128, 128)]], bufs[s], gsems[s])
            if prev is not None:
                ps, pg, pch = prev
                pg.wait()
                pend_out[ps] = pltpu.async_copy(
                    bufs[ps], xj_hbm.at[pl.ds(base + pch * 128, 128)],
                    osems[ps])
            prev = (s, g, ch)
        ps, pg, pch = prev
        pg.wait()
        pend_out[ps] = pltpu.async_copy(
            bufs[ps], xj_hbm.at[pl.ds(base + pch * 128, 128)], osems[ps])
        for s in range(3):
            if pend_out[s] is not None:
                pend_out[s].wait()

    return stage_b


def _stage_c_body(xj_ref, xi_ref, y_ref, wb_ref, q_ref, p1_ref, p2_ref):
    pt = xi_ref.shape[0]
    c_pad = xi_ref.shape[1]
    xj = xj_ref[...].reshape(pt, KNN, c_pad)
    diff = (xj - xi_ref[...][:, None, :]).astype(jnp.bfloat16)
    wb = wb_ref[...].astype(jnp.bfloat16)
    gm = lax.dot_general(diff.reshape(pt * KNN, c_pad), wb,
                         (((1,), (0,)), ((), ())),
                         preferred_element_type=jnp.float32)
    gm = gm.reshape(pt, KNN, wb.shape[1])
    y = y_ref[...]
    s1 = jnp.sum(gm, axis=1)
    s2 = jnp.sum(gm * gm, axis=1)
    k = float(KNN)
    q_ref[...] = y + jnp.max(gm, axis=1)
    p1_ref[0] = jnp.sum(k * y + s1, axis=0, keepdims=True)
    p2_ref[0] = jnp.sum(k * y * y + 2.0 * y * s1 + s2, axis=0,
                        keepdims=True)


def _stage_c(xj, x2f, yf, wb_pad):
    n_total, c_pad = x2f.shape
    g_out = yf.shape[1]
    pt = PT_TILE
    n_tiles = n_total // pt
    return pl.pallas_call(
        _stage_c_body,
        grid=(n_tiles,),
        in_specs=[
            pl.BlockSpec((pt * KNN, c_pad), lambda t: (t, 0)),
            pl.BlockSpec((pt, c_pad), lambda t: (t, 0)),
            pl.BlockSpec((pt, g_out), lambda t: (t, 0)),
            pl.BlockSpec((c_pad, g_out), lambda t: (0, 0)),
        ],
        out_specs=[
            pl.BlockSpec((pt, g_out), lambda t: (t, 0)),
            pl.BlockSpec((1, 1, g_out), lambda t: (t, 0, 0)),
            pl.BlockSpec((1, 1, g_out), lambda t: (t, 0, 0)),
        ],
        out_shape=[
            jax.ShapeDtypeStruct((n_total, g_out), jnp.float32),
            jax.ShapeDtypeStruct((n_tiles, 1, g_out), jnp.float32),
            jax.ShapeDtypeStruct((n_tiles, 1, g_out), jnp.float32),
        ],
    )(xj, x2f, yf, wb_pad)


def _stage_f_body(n_total, q_ref, p1_ref, p2_ref, g_ref, be_ref, f_ref):
    f_ref[...] = _bn_finish(q_ref[...], p1_ref, p2_ref, g_ref, be_ref,
                            n_total)


def _stage_f(q, p1, p2, gamma, beta):
    n_total, g_out = q.shape
    return pl.pallas_call(
        functools.partial(_stage_f_body, n_total),
        out_shape=jax.ShapeDtypeStruct((n_total, g_out), jnp.float32),
    )(q, p1, p2, gamma, beta)


def _wab(w, c_in):
    wa = jnp.transpose(w[:, :c_in], (1, 0))
    wb = jnp.transpose(w[:, c_in:], (1, 0))
    return wa, wb


def _gather_and_edges(idx, x2, wb, y):
    bsz, n_pts, c_pad = x2.shape
    g_out = wb.shape[1]
    c_in = wb.shape[0]
    n_total = bsz * n_pts
    wb_pad = jnp.concatenate(
        [wb, jnp.zeros((c_pad - c_in, g_out), jnp.float32)], axis=0)
    idx_w = jnp.transpose(idx, (0, 2, 1)).reshape(
        NUM_WORKERS, (n_total * KNN) // NUM_WORKERS)
    x2f = x2.reshape(n_total, c_pad)
    xj = _make_stage_b(n_total, c_pad)(x2f, idx_w)
    return _stage_c(xj, x2f, y.reshape(n_total, g_out), wb_pad)


def kernel(inputs, W0, b0, g0, be0, W1, b1, g1, be1, W2, b2, g2, be2):
    xt0 = jnp.transpose(inputs[..., 0], (0, 2, 1))
    bsz, n_pts, c0 = xt0.shape
    n_total = bsz * n_pts

    wa0, wb0 = _wab(W0, c0)
    cp1 = 128
    idx0, y0, x2_0 = _stage_a1(xt0, wa0, b0[None, :], cp1)
    q0, p10, p20 = _gather_and_edges(idx0, x2_0, wb0, y0)

    c1 = c0 + 64
    wa1, wb1 = _wab(W1, c1)
    idx1, y1, x2_1 = _stage_a23(x2_0, q0.reshape(bsz, n_pts, -1), p10, p20,
                                g0[None, :], be0[None, :], wa1, b1[None, :],
                                c0, 128)
    q1, p11, p21 = _gather_and_edges(idx1, x2_1, wb1, y1)

    c2 = c1 + 64
    wa2, wb2 = _wab(W2, c2)
    idx2, y2, x2_2 = _stage_a23(x2_1, q1.reshape(bsz, n_pts, -1), p11, p21,
                                g1[None, :], be1[None, :], wa2, b2[None, :],
                                c1, 256)
    q2, p12, p22 = _gather_and_edges(idx2, x2_2, wb2, y2)

    f2 = _stage_f(q2, p12, p22, g2[None, :], be2[None, :])
    out = jnp.concatenate(
        [f2.reshape(bsz, n_pts, 64), x2_2[:, :, :c2]], axis=-1)
    return jnp.transpose(out, (0, 2, 1))[..., None]


# speedup vs baseline: 1.0252x; 1.0047x over previous
import functools

import jax
import jax.numpy as jnp
from jax import lax
from jax.experimental import pallas as pl
from jax.experimental.pallas import tpu as pltpu
from jax.experimental.pallas import tpu_sc as plsc

KNN = 20
EPSILON = 1e-5
ROW_TILE = 1024
PT_TILE = 256
NEG_BIG = -1e30
NUM_WORKERS = 32


def _topk_y_and_outputs(n_pts, base, xr, xa, wa_ref, b_ref,
                        idx_ref, y_ref, x2_ref, c_pad):
    xrb = xr.astype(jnp.bfloat16)
    xab = xa.astype(jnp.bfloat16)
    dot = lax.dot_general(xrb, xab, (((1,), (1,)), ((), ())),
                          preferred_element_type=jnp.float32)
    sq_r = jnp.sum(xr * xr, axis=1, keepdims=True)
    sq_a = jnp.sum(xa * xa, axis=1)[None, :]
    d = 2.0 * dot - sq_r - sq_a
    n_iota = lax.broadcasted_iota(jnp.int32, d.shape, 1)
    cols = []
    for _ in range(KNN):
        m = jnp.max(d, axis=1, keepdims=True)
        eq = d == m
        cand = jnp.where(eq, n_iota, n_pts)
        jmin = jnp.min(cand, axis=1)
        cols.append(jmin[None, :] + base)
        d = jnp.where(eq, NEG_BIG, d)
    idx_ref[0] = jnp.concatenate(cols, axis=0)
    y_ref[0] = lax.dot_general(
        xrb, wa_ref[...].astype(jnp.bfloat16), (((1,), (0,)), ((), ())),
        preferred_element_type=jnp.float32) + b_ref[...]
    pad = c_pad - xr.shape[1]
    if pad:
        x2_ref[0] = jnp.concatenate(
            [xr, jnp.zeros((xr.shape[0], pad), jnp.float32)], axis=1)
    else:
        x2_ref[0] = xr


def _stage_a1_body(n_pts, c_pad, xr_ref, xa_ref, wa_ref, b_ref,
                   idx_ref, y_ref, x2_ref):
    base = pl.program_id(0) * n_pts
    _topk_y_and_outputs(n_pts, base, xr_ref[0], xa_ref[0], wa_ref, b_ref,
                        idx_ref, y_ref, x2_ref, c_pad)


def _stage_a1(xt, wa, bias, c_pad):
    bsz, n_pts, c_in = xt.shape
    g_out = wa.shape[1]
    r = ROW_TILE
    return pl.pallas_call(
        functools.partial(_stage_a1_body, n_pts, c_pad),
        grid=(bsz, n_pts // r),
        in_specs=[
            pl.BlockSpec((1, r, c_in), lambda b, t: (b, t, 0)),
            pl.BlockSpec((1, n_pts, c_in), lambda b, t: (b, 0, 0)),
            pl.BlockSpec((c_in, g_out), lambda b, t: (0, 0)),
            pl.BlockSpec((1, g_out), lambda b, t: (0, 0)),
        ],
        out_specs=[
            pl.BlockSpec((1, KNN, r), lambda b, t: (b, 0, t)),
            pl.BlockSpec((1, r, g_out), lambda b, t: (b, t, 0)),
            pl.BlockSpec((1, r, c_pad), lambda b, t: (b, t, 0)),
        ],
        out_shape=[
            jax.ShapeDtypeStruct((bsz, KNN, n_pts), jnp.int32),
            jax.ShapeDtypeStruct((bsz, n_pts, g_out), jnp.float32),
            jax.ShapeDtypeStruct((bsz, n_pts, c_pad), jnp.float32),
        ],
    )(xt, xt, wa, bias)


def _bn_finish(q, p1_ref, p2_ref, g_ref, be_ref, n_total):
    cnt = float(KNN) * n_total
    mean = jnp.sum(p1_ref[...], axis=0) / cnt
    e2 = jnp.sum(p2_ref[...], axis=0) / cnt
    var = e2 - mean * mean
    scale = lax.rsqrt(var + EPSILON) * g_ref[...]
    return jnp.maximum((q - mean) * scale + be_ref[...], 0.0)


def _stage_a23_body(n_pts, n_total, c_prev, c_pad, xp_r_ref, xp_a_ref,
                    q_r_ref, q_a_ref, p1_ref, p2_ref, g_ref, be_ref,
                    wa_ref, b_ref, idx_ref, y_ref, x2_ref):
    base = pl.program_id(0) * n_pts
    f_a = _bn_finish(q_a_ref[0], p1_ref, p2_ref, g_ref, be_ref, n_total)
    f_r = _bn_finish(q_r_ref[0], p1_ref, p2_ref, g_ref, be_ref, n_total)
    xa = jnp.concatenate([f_a, xp_a_ref[0][:, :c_prev]], axis=1)
    xr = jnp.concatenate([f_r, xp_r_ref[0][:, :c_prev]], axis=1)
    _topk_y_and_outputs(n_pts, base, xr, xa, wa_ref, b_ref,
                        idx_ref, y_ref, x2_ref, c_pad)


def _stage_a23(x2_prev, q, p1, p2, gamma, beta, wa, bias, c_prev, c_pad):
    bsz, n_pts, cp_prev = x2_prev.shape
    g_out = q.shape[-1]
    n_total = bsz * n_pts
    r = ROW_TILE
    n_tiles = p1.shape[0]
    return pl.pallas_call(
        functools.partial(_stage_a23_body, n_pts, n_total, c_prev, c_pad),
        grid=(bsz, n_pts // r),
        in_specs=[
            pl.BlockSpec((1, r, cp_prev), lambda b, t: (b, t, 0)),
            pl.BlockSpec((1, n_pts, cp_prev), lambda b, t: (b, 0, 0)),
            pl.BlockSpec((1, r, g_out), lambda b, t: (b, t, 0)),
            pl.BlockSpec((1, n_pts, g_out), lambda b, t: (b, 0, 0)),
            pl.BlockSpec((n_tiles, 1, g_out), lambda b, t: (0, 0, 0)),
            pl.BlockSpec((n_tiles, 1, g_out), lambda b, t: (0, 0, 0)),
            pl.BlockSpec((1, g_out), lambda b, t: (0, 0)),
            pl.BlockSpec((1, g_out), lambda b, t: (0, 0)),
            pl.BlockSpec((c_prev + g_out, g_out), lambda b, t: (0, 0)),
            pl.BlockSpec((1, g_out), lambda b, t: (0, 0)),
        ],
        out_specs=[
            pl.BlockSpec((1, KNN, r), lambda b, t: (b, 0, t)),
            pl.BlockSpec((1, r, g_out), lambda b, t: (b, t, 0)),
            pl.BlockSpec((1, r, c_pad), lambda b, t: (b, t, 0)),
        ],
        out_shape=[
            jax.ShapeDtypeStruct((bsz, KNN, n_pts), jnp.int32),
            jax.ShapeDtypeStruct((bsz, n_pts, g_out), jnp.float32),
            jax.ShapeDtypeStruct((bsz, n_pts, c_pad), jnp.float32),
        ],
    )(x2_prev, x2_prev, q, q, p1, p2, gamma, beta, wa, bias)


def _make_stage_b(n_total, c_pad):
    rows = (n_total * KNN) // NUM_WORKERS
    nch = rows // 128
    mesh = plsc.VectorSubcoreMesh(core_axis_name="c", subcore_axis_name="s")

    @functools.partial(
        pl.kernel,
        out_type=jax.ShapeDtypeStruct((n_total * KNN, c_pad), jnp.float32),
        mesh=mesh,
        compiler_params=pltpu.CompilerParams(needs_layout_passes=False),
        scratch_types=[
            pltpu.VMEM((rows,), jnp.int32),
            pltpu.VMEM((128, c_pad), jnp.float32),
            pltpu.VMEM((128, c_pad), jnp.float32),
            pltpu.VMEM((128, c_pad), jnp.float32),
            pltpu.SemaphoreType.DMA,
            pltpu.SemaphoreType.DMA,
            pltpu.SemaphoreType.DMA,
            pltpu.SemaphoreType.DMA,
            pltpu.SemaphoreType.DMA,
            pltpu.SemaphoreType.DMA,
        ],
    )
    def stage_b(x_hbm, idx_hbm, xj_hbm, idx_v, buf0, buf1, buf2,
                g0, g1, g2, o0, o1, o2):
        wid = lax.axis_index("s") * 2 + lax.axis_index("c")
        base = wid * rows
        pltpu.sync_copy(idx_hbm.at[wid], idx_v)
        bufs = (buf0, buf1, buf2)
        gsems = (g0, g1, g2)
        osems = (o0, o1, o2)
        pend_out = [None, None, None]
        prev = None
        for ch in range(nch):
            s = ch % 3
            if pend_out[s] is not None:
                pend_out[s].wait()
                pend_out[s] = None
            g = pltpu.async_copy(
                x_hbm.at[idx_v.at[pl.ds(ch * 128, 128)]], bufs[s], gsems[s])
            if prev is not None:
                ps, pg, pch = prev
                pg.wait()
                pend_out[ps] = pltpu.async_copy(
                    bufs[ps], xj_hbm.at[pl.ds(base + pch * 128, 128)],
                    osems[ps])
            prev = (s, g, ch)
        ps, pg, pch = prev
        pg.wait()
        pend_out[ps] = pltpu.async_copy(
            bufs[ps], xj_hbm.at[pl.ds(base + pch * 128, 128)], osems[ps])
        for s in range(3):
            if pend_out[s] is not None:
                pend_out[s].wait()

    return stage_b


def _stage_c_body(xj_ref, xi_ref, y_ref, wb_ref, q_ref, p1_ref, p2_ref):
    pt = xi_ref.shape[0]
    c_pad = xi_ref.shape[1]
    xj = xj_ref[...].reshape(pt, KNN, c_pad)
    diff = (xj - xi_ref[...][:, None, :]).astype(jnp.bfloat16)
    wb = wb_ref[...].astype(jnp.bfloat16)
    gm = lax.dot_general(diff.reshape(pt * KNN, c_pad), wb,
                         (((1,), (0,)), ((), ())),
                         preferred_element_type=jnp.float32)
    gm = gm.reshape(pt, KNN, wb.shape[1])
    y = y_ref[...]
    s1 = jnp.sum(gm, axis=1)
    s2 = jnp.sum(gm * gm, axis=1)
    k = float(KNN)
    q_ref[...] = y + jnp.max(gm, axis=1)
    p1_ref[0] = jnp.sum(k * y + s1, axis=0, keepdims=True)
    p2_ref[0] = jnp.sum(k * y * y + 2.0 * y * s1 + s2, axis=0,
                        keepdims=True)


def _stage_c(xj, x2f, yf, wb_pad):
    n_total, c_pad = x2f.shape
    g_out = yf.shape[1]
    pt = PT_TILE
    n_tiles = n_total // pt
    return pl.pallas_call(
        _stage_c_body,
        grid=(n_tiles,),
        in_specs=[
            pl.BlockSpec((pt * KNN, c_pad), lambda t: (t, 0)),
            pl.BlockSpec((pt, c_pad), lambda t: (t, 0)),
            pl.BlockSpec((pt, g_out), lambda t: (t, 0)),
            pl.BlockSpec((c_pad, g_out), lambda t: (0, 0)),
        ],
        out_specs=[
            pl.BlockSpec((pt, g_out), lambda t: (t, 0)),
            pl.BlockSpec((1, 1, g_out), lambda t: (t, 0, 0)),
            pl.BlockSpec((1, 1, g_out), lambda t: (t, 0, 0)),
        ],
        out_shape=[
            jax.ShapeDtypeStruct((n_total, g_out), jnp.float32),
            jax.ShapeDtypeStruct((n_tiles, 1, g_out), jnp.float32),
            jax.ShapeDtypeStruct((n_tiles, 1, g_out), jnp.float32),
        ],
    )(xj, x2f, yf, wb_pad)


def _stage_f_body(n_total, q_ref, p1_ref, p2_ref, g_ref, be_ref, f_ref):
    f_ref[...] = _bn_finish(q_ref[...], p1_ref, p2_ref, g_ref, be_ref,
                            n_total)


def _stage_f(q, p1, p2, gamma, beta):
    n_total, g_out = q.shape
    return pl.pallas_call(
        functools.partial(_stage_f_body, n_total),
        out_shape=jax.ShapeDtypeStruct((n_total, g_out), jnp.float32),
    )(q, p1, p2, gamma, beta)


def _wab(w, c_in):
    wa = jnp.transpose(w[:, :c_in], (1, 0))
    wb = jnp.transpose(w[:, c_in:], (1, 0))
    return wa, wb


def _gather_and_edges(idx, x2, wb, y):
    bsz, n_pts, c_pad = x2.shape
    g_out = wb.shape[1]
    c_in = wb.shape[0]
    n_total = bsz * n_pts
    wb_pad = jnp.concatenate(
        [wb, jnp.zeros((c_pad - c_in, g_out), jnp.float32)], axis=0)
    idx_w = jnp.transpose(idx, (0, 2, 1)).reshape(
        NUM_WORKERS, (n_total * KNN) // NUM_WORKERS)
    x2f = x2.reshape(n_total, c_pad)
    xj = _make_stage_b(n_total, c_pad)(x2f, idx_w)
    return _stage_c(xj, x2f, y.reshape(n_total, g_out), wb_pad)


def kernel(inputs, W0, b0, g0, be0, W1, b1, g1, be1, W2, b2, g2, be2):
    xt0 = jnp.transpose(inputs[..., 0], (0, 2, 1))
    bsz, n_pts, c0 = xt0.shape
    n_total = bsz * n_pts

    wa0, wb0 = _wab(W0, c0)
    cp1 = 128
    idx0, y0, x2_0 = _stage_a1(xt0, wa0, b0[None, :], cp1)
    q0, p10, p20 = _gather_and_edges(idx0, x2_0, wb0, y0)

    c1 = c0 + 64
    wa1, wb1 = _wab(W1, c1)
    idx1, y1, x2_1 = _stage_a23(x2_0, q0.reshape(bsz, n_pts, -1), p10, p20,
                                g0[None, :], be0[None, :], wa1, b1[None, :],
                                c0, 128)
    q1, p11, p21 = _gather_and_edges(idx1, x2_1, wb1, y1)

    c2 = c1 + 64
    wa2, wb2 = _wab(W2, c2)
    idx2, y2, x2_2 = _stage_a23(x2_1, q1.reshape(bsz, n_pts, -1), p11, p21,
                                g1[None, :], be1[None, :], wa2, b2[None, :],
                                c1, 256)
    q2, p12, p22 = _gather_and_edges(idx2, x2_2, wb2, y2)

    f2 = _stage_f(q2, p12, p22, g2[None, :], be2[None, :])
    out = jnp.concatenate(
        [f2.reshape(bsz, n_pts, 64), x2_2[:, :, :c2]], axis=-1)
    return jnp.transpose(out, (0, 2, 1))[..., None]
